# baseline (speedup 1.0000x reference)
import functools

import jax
import jax.numpy as jnp
from jax.experimental import pallas as pl
from jax.experimental.pallas import tpu as pltpu

_K = 5
_CAND = 12
_HARV = 5
_PAD = 8
_NEG = float("-inf")
_IMAX = 2**30
_GATH = 40
_NCPS = 16


def _normalize_q(q):
    return q / jnp.maximum(
        jnp.sqrt(jnp.sum(q * q, axis=1, keepdims=True)), 1e-12)


def _extract_topk(cs, ci, k):
    outs_s, outs_i = [], []
    for j in range(k):
        m = jnp.max(cs, axis=1, keepdims=True)
        hit = cs == m
        idx = jnp.min(jnp.where(hit, ci, _IMAX), axis=1, keepdims=True)
        outs_s.append(m)
        outs_i.append(idx)
        if j < k - 1:
            cs = jnp.where(ci == idx, _NEG, cs)
    return outs_s, outs_i


def _scan_kernel(q_ref, e_ref, cand_ref, run_s, ss_buf, *, blk, nblk):
    i = pl.program_id(0)

    @pl.when(i == 0)
    def _init():
        run_s[...] = jnp.full((16, _CAND), _NEG, jnp.float32)
        cand_ref[...] = jnp.full((16, _CAND), _IMAX, jnp.int32)

    @pl.when(i < nblk)
    def _score():
        qn = _normalize_q(q_ref[...])

        e = e_ref[...]
        e_bf = e.astype(jnp.bfloat16)
        e2_bf = e_bf * e_bf
        s_raw = jax.lax.dot_general(
            qn.astype(jnp.bfloat16), e_bf, (((1,), (1,)), ((), ())),
            preferred_element_type=jnp.float32)
        ssb = jax.lax.dot_general(
            jnp.ones((16, e.shape[1]), jnp.bfloat16), e2_bf,
            (((1,), (1,)), ((), ())),
            preferred_element_type=jnp.float32)
        s_sel = s_raw * jax.lax.rsqrt(jnp.maximum(ssb, 1e-12))
        ss_buf[pl.ds(i % 2, 1), :, :] = s_sel[None]

    @pl.when(i > 0)
    def _tournament():
        b = i - 1
        prev = ss_buf[pl.ds((i + 1) % 2, 1), :, :][0]

        lane = jax.lax.broadcasted_iota(jnp.int32, (16, 128), 1)
        nslab = blk // 128
        tail = blk - nslab * 128
        leaves = []
        for j in range(nslab + (1 if tail else 0)):
            if j < nslab:
                v = prev[:, j * 128:(j + 1) * 128]
            else:
                v = jnp.concatenate(
                    [prev[:, nslab * 128:],
                     jnp.full((16, 128 - tail), _NEG, jnp.float32)], axis=1)
            leaves.append((v, lane + (b * blk + j * 128)))


        def _pair(la, lb):
            (va, xa), (vb, xb) = la, lb
            c = va >= vb
            return (jnp.maximum(va, vb), jnp.where(c, xa, xb),
                    jnp.minimum(va, vb), jnp.where(c, xb, xa))

        def _comb(na, nb):
            a1, ai1, a2, ai2 = na
            b1, bi1, b2, bi2 = nb
            c = a1 >= b1
            lo = jnp.minimum(a1, b1)
            loi = jnp.where(c, bi1, ai1)
            ws = jnp.where(c, a2, b2)
            wsi = jnp.where(c, ai2, bi2)
            c2 = lo >= ws
            return (jnp.maximum(a1, b1), jnp.where(c, ai1, bi1),
                    jnp.maximum(lo, ws), jnp.where(c2, loi, wsi))

        nodes = [_pair(leaves[a], leaves[a + 1])
                 for a in range(0, len(leaves) - 1, 2)]
        if len(leaves) % 2:
            v, x = leaves[-1]
            nodes.append((v, x, jnp.full((16, 128), _NEG, jnp.float32),
                          jnp.full((16, 128), _IMAX, jnp.int32)))
        while len(nodes) > 1:
            nxt = [_comb(nodes[a], nodes[a + 1])
                   for a in range(0, len(nodes) - 1, 2)]
            if len(nodes) % 2:
                nxt.append(nodes[-1])
            nodes = nxt
        t1, t1i, t2, t2i = nodes[0]

        cs = jnp.concatenate([run_s[...], t1, t2], axis=1)
        ci = jnp.concatenate([cand_ref[...], t1i, t2i], axis=1)
        ms, mi = _extract_topk(cs, ci, _CAND)
        run_s[...] = jnp.concatenate(ms, axis=1)
        cand_ref[...] = jnp.concatenate(mi, axis=1)


def _rescore_kernel(idx_ref, q_ref, *refs):
    e_refs = refs[:_NCPS]
    out_i_ref, out_s_ref = refs[_NCPS], refs[_NCPS + 1]
    c = pl.program_id(0)

    @pl.when(c == 0)
    def _init():
        out_s_ref[...] = jnp.full((16, _PAD), _NEG, jnp.float32)
        out_i_ref[...] = jnp.full((16, _PAD), _IMAX, jnp.int32)

    qn = _normalize_q(q_ref[...])

    all_s, all_i = [out_s_ref[...]], [out_i_ref[...]]
    iota = jax.lax.broadcasted_iota(jnp.int32, (16, _GATH), 1)
    for j in range(_NCPS):
        e = e_refs[j][...]
        ss = jnp.sum(e * e, axis=1, keepdims=True)
        en = e * (1.0 / jnp.maximum(jnp.sqrt(ss), 1e-12))
        s = jax.lax.dot_general(
            qn, en, (((1,), (1,)), ((), ())),
            preferred_element_type=jnp.float32)
        row0 = (idx_ref[c * _NCPS + j] // _GATH) * _GATH
        all_s.append(s)
        all_i.append(row0 + iota)

    cs = jnp.concatenate(all_s, axis=1)
    ci = jnp.concatenate(all_i, axis=1)
    fs, fi = _extract_topk(cs, ci, _K)
    out_s_ref[...] = jnp.concatenate(
        fs + [jnp.full((16, _PAD - _K), _NEG, jnp.float32)], axis=1)
    out_i_ref[...] = jnp.concatenate(
        fi + [jnp.full((16, _PAD - _K), _IMAX, jnp.int32)], axis=1)


def kernel(query_embedding, evidence_embeddings, top_k):
    del top_k
    n, d = evidence_embeddings.shape
    blk = 20000 if n % 20000 == 0 else n
    nblk = n // blk

    cand = pl.pallas_call(
        functools.partial(_scan_kernel, blk=blk, nblk=nblk),
        grid=(nblk + 1,),
        in_specs=[
            pl.BlockSpec((16, d), lambda i: (0, 0)),
            pl.BlockSpec((blk, d), lambda i: (jnp.minimum(i, nblk - 1), 0)),
        ],
        out_specs=pl.BlockSpec((16, _CAND), lambda i: (0, 0)),
        out_shape=jax.ShapeDtypeStruct((16, _CAND), jnp.int32),
        scratch_shapes=[pltpu.VMEM((16, _CAND), jnp.float32),
                        pltpu.VMEM((2, 16, blk), jnp.float32)],
        compiler_params=pltpu.CompilerParams(
            dimension_semantics=("arbitrary",)),
    )(query_embedding, evidence_embeddings)

    ncand = 16 * _CAND
    flat = cand.reshape(-1)

    out_i, out_s = pl.pallas_call(
        _rescore_kernel,
        grid_spec=pltpu.PrefetchScalarGridSpec(
            num_scalar_prefetch=1,
            grid=(ncand // _NCPS,),
            in_specs=[pl.BlockSpec((16, d), lambda c, s: (0, 0))] + [
                pl.BlockSpec(
                    (_GATH, d),
                    functools.partial(
                        lambda c, s, j: (s[c * _NCPS + j] // _GATH, 0), j=j))
                for j in range(_NCPS)
            ],
            out_specs=[
                pl.BlockSpec((16, _PAD), lambda c, s: (0, 0)),
                pl.BlockSpec((16, _PAD), lambda c, s: (0, 0)),
            ],
        ),
        out_shape=[
            jax.ShapeDtypeStruct((16, _PAD), jnp.int32),
            jax.ShapeDtypeStruct((16, _PAD), jnp.float32),
        ],
        compiler_params=pltpu.CompilerParams(
            dimension_semantics=("arbitrary",)),
    )(flat, query_embedding,
      *([evidence_embeddings] * _NCPS))

    return out_i[:, :_K], out_s[:, :_K]


# speedup vs baseline: 1.0933x; 1.0933x over previous
import functools

import jax
import jax.numpy as jnp
from jax.experimental import pallas as pl
from jax.experimental.pallas import tpu as pltpu

_K = 5
_CAND = 12
_HARV = 5
_PAD = 8
_NEG = float("-inf")
_IMAX = 2**30
_GATH = 40
_NCPS = 16


def _normalize_q(q):
    return q / jnp.maximum(
        jnp.sqrt(jnp.sum(q * q, axis=1, keepdims=True)), 1e-12)


def _extract_topk(cs, ci, k):
    outs_s, outs_i = [], []
    for j in range(k):
        m = jnp.max(cs, axis=1, keepdims=True)
        hit = cs == m
        idx = jnp.min(jnp.where(hit, ci, _IMAX), axis=1, keepdims=True)
        outs_s.append(m)
        outs_i.append(idx)
        if j < k - 1:
            cs = jnp.where(ci == idx, _NEG, cs)
    return outs_s, outs_i


def _scan_kernel(q_ref, e_ref, cand_ref, run_s, *, blk, nblk):
    i = pl.program_id(0)

    @pl.when(i == 0)
    def _init():
        run_s[...] = jnp.full((16, _CAND), _NEG, jnp.float32)
        cand_ref[...] = jnp.full((16, _CAND), _IMAX, jnp.int32)

    qn = _normalize_q(q_ref[...])

    e = e_ref[...]
    e_bf = e.astype(jnp.bfloat16)
    e2_bf = e_bf * e_bf
    s_raw = jax.lax.dot_general(
        qn.astype(jnp.bfloat16), e_bf, (((1,), (1,)), ((), ())),
        preferred_element_type=jnp.float32)
    ssb = jax.lax.dot_general(
        jnp.ones((16, e.shape[1]), jnp.bfloat16), e2_bf,
        (((1,), (1,)), ((), ())),
        preferred_element_type=jnp.float32)
    s_sel = s_raw * jax.lax.rsqrt(jnp.maximum(ssb, 1e-12))

    lane = jax.lax.broadcasted_iota(jnp.int32, (16, 128), 1)
    nslab = blk // 128
    tail = blk - nslab * 128
    leaves = []
    for j in range(nslab + (1 if tail else 0)):
        if j < nslab:
            v = s_sel[:, j * 128:(j + 1) * 128]
        else:
            v = jnp.concatenate(
                [s_sel[:, nslab * 128:],
                 jnp.full((16, 128 - tail), _NEG, jnp.float32)], axis=1)
        leaves.append((v, lane + (i * blk + j * 128)))

    def _pair(la, lb):
        (va, xa), (vb, xb) = la, lb
        c = va >= vb
        return (jnp.maximum(va, vb), jnp.where(c, xa, xb),
                jnp.minimum(va, vb), jnp.where(c, xb, xa))

    def _comb(na, nb):
        a1, ai1, a2, ai2 = na
        b1, bi1, b2, bi2 = nb
        c = a1 >= b1
        lo = jnp.minimum(a1, b1)
        loi = jnp.where(c, bi1, ai1)
        ws = jnp.where(c, a2, b2)
        wsi = jnp.where(c, ai2, bi2)
        c2 = lo >= ws
        return (jnp.maximum(a1, b1), jnp.where(c, ai1, bi1),
                jnp.maximum(lo, ws), jnp.where(c2, loi, wsi))

    nodes = [_pair(leaves[a], leaves[a + 1])
             for a in range(0, len(leaves) - 1, 2)]
    if len(leaves) % 2:
        v, x = leaves[-1]
        nodes.append((v, x, jnp.full((16, 128), _NEG, jnp.float32),
                      jnp.full((16, 128), _IMAX, jnp.int32)))
    while len(nodes) > 1:
        nxt = [_comb(nodes[a], nodes[a + 1])
               for a in range(0, len(nodes) - 1, 2)]
        if len(nodes) % 2:
            nxt.append(nodes[-1])
        nodes = nxt
    t1, t1i, t2, t2i = nodes[0]

    cs = jnp.concatenate([run_s[...], t1, t2], axis=1)
    ci = jnp.concatenate([cand_ref[...], t1i, t2i], axis=1)
    ms, mi = _extract_topk(cs, ci, _CAND)
    run_s[...] = jnp.concatenate(ms, axis=1)
    cand_ref[...] = jnp.concatenate(mi, axis=1)


def _rescore_kernel(idx_ref, q_ref, *refs):
    e_refs = refs[:_NCPS]
    out_i_ref, out_s_ref = refs[_NCPS], refs[_NCPS + 1]
    c = pl.program_id(0)

    @pl.when(c == 0)
    def _init():
        out_s_ref[...] = jnp.full((16, _PAD), _NEG, jnp.float32)
        out_i_ref[...] = jnp.full((16, _PAD), _IMAX, jnp.int32)

    qn = _normalize_q(q_ref[...])

    all_s, all_i = [out_s_ref[...]], [out_i_ref[...]]
    iota = jax.lax.broadcasted_iota(jnp.int32, (16, _GATH), 1)
    for j in range(_NCPS):
        e = e_refs[j][...]
        ss = jnp.sum(e * e, axis=1, keepdims=True)
        en = e * (1.0 / jnp.maximum(jnp.sqrt(ss), 1e-12))
        s = jax.lax.dot_general(
            qn, en, (((1,), (1,)), ((), ())),
            preferred_element_type=jnp.float32)
        row0 = (idx_ref[c * _NCPS + j] // _GATH) * _GATH
        all_s.append(s)
        all_i.append(row0 + iota)

    cs = jnp.concatenate(all_s, axis=1)
    ci = jnp.concatenate(all_i, axis=1)
    fs, fi = _extract_topk(cs, ci, _K)
    out_s_ref[...] = jnp.concatenate(
        fs + [jnp.full((16, _PAD - _K), _NEG, jnp.float32)], axis=1)
    out_i_ref[...] = jnp.concatenate(
        fi + [jnp.full((16, _PAD - _K), _IMAX, jnp.int32)], axis=1)


def kernel(query_embedding, evidence_embeddings, top_k):
    del top_k
    n, d = evidence_embeddings.shape
    blk = 20000 if n % 20000 == 0 else n
    nblk = n // blk

    cand = pl.pallas_call(
        functools.partial(_scan_kernel, blk=blk, nblk=nblk),
        grid=(nblk,),
        in_specs=[
            pl.BlockSpec((16, d), lambda i: (0, 0)),
            pl.BlockSpec((blk, d), lambda i: (i, 0)),
        ],
        out_specs=pl.BlockSpec((16, _CAND), lambda i: (0, 0)),
        out_shape=jax.ShapeDtypeStruct((16, _CAND), jnp.int32),
        scratch_shapes=[pltpu.VMEM((16, _CAND), jnp.float32)],
        compiler_params=pltpu.CompilerParams(
            dimension_semantics=("arbitrary",)),
    )(query_embedding, evidence_embeddings)

    ncand = 16 * _CAND
    flat = cand.reshape(-1)

    out_i, out_s = pl.pallas_call(
        _rescore_kernel,
        grid_spec=pltpu.PrefetchScalarGridSpec(
            num_scalar_prefetch=1,
            grid=(ncand // _NCPS,),
            in_specs=[pl.BlockSpec((16, d), lambda c, s: (0, 0))] + [
                pl.BlockSpec(
                    (_GATH, d),
                    functools.partial(
                        lambda c, s, j: (s[c * _NCPS + j] // _GATH, 0), j=j))
                for j in range(_NCPS)
            ],
            out_specs=[
                pl.BlockSpec((16, _PAD), lambda c, s: (0, 0)),
                pl.BlockSpec((16, _PAD), lambda c, s: (0, 0)),
            ],
        ),
        out_shape=[
            jax.ShapeDtypeStruct((16, _PAD), jnp.int32),
            jax.ShapeDtypeStruct((16, _PAD), jnp.float32),
        ],
        compiler_params=pltpu.CompilerParams(
            dimension_semantics=("arbitrary",)),
    )(flat, query_embedding,
      *([evidence_embeddings] * _NCPS))

    return out_i[:, :_K], out_s[:, :_K]


# speedup vs baseline: 1.6820x; 1.5384x over previous
import functools

import jax
import jax.numpy as jnp
from jax.experimental import pallas as pl
from jax.experimental.pallas import tpu as pltpu

_K = 5
_CAND = 12
_HARV = 5
_PAD = 8
_NEG = float("-inf")
_IMAX = 2**30
_GATH = 40
_NCPS = 16
_ACC = 6


def _normalize_q(q):
    return q / jnp.maximum(
        jnp.sqrt(jnp.sum(q * q, axis=1, keepdims=True)), 1e-12)


def _extract_topk(cs, ci, k):
    outs_s, outs_i = [], []
    for j in range(k):
        m = jnp.max(cs, axis=1, keepdims=True)
        hit = cs == m
        idx = jnp.min(jnp.where(hit, ci, _IMAX), axis=1, keepdims=True)
        outs_s.append(m)
        outs_i.append(idx)
        if j < k - 1:
            cs = jnp.where(ci == idx, _NEG, cs)
    return outs_s, outs_i


def _scan_kernel(q_ref, e_ref, cand_ref, acc_s, acc_i, *, blk, nblk):
    i = pl.program_id(0)

    @pl.when(i == 0)
    def _init():
        acc_s[...] = jnp.full((16, _ACC * 128), _NEG, jnp.float32)
        acc_i[...] = jnp.full((16, _ACC * 128), _IMAX, jnp.int32)

    qn = _normalize_q(q_ref[...])

    e = e_ref[...]
    e_bf = e.astype(jnp.bfloat16)
    e2_bf = e_bf * e_bf
    s_raw = jax.lax.dot_general(
        qn.astype(jnp.bfloat16), e_bf, (((1,), (1,)), ((), ())),
        preferred_element_type=jnp.float32)
    ssb = jax.lax.dot_general(
        jnp.ones((16, e.shape[1]), jnp.bfloat16), e2_bf,
        (((1,), (1,)), ((), ())),
        preferred_element_type=jnp.float32)
    s_sel = s_raw * jax.lax.rsqrt(jnp.maximum(ssb, 1e-12))

    lane = jax.lax.broadcasted_iota(jnp.int32, (16, 128), 1)
    nslab = blk // 128
    tail = blk - nslab * 128
    leaves = []
    for j in range(nslab + (1 if tail else 0)):
        if j < nslab:
            v = s_sel[:, j * 128:(j + 1) * 128]
        else:
            v = jnp.concatenate(
                [s_sel[:, nslab * 128:],
                 jnp.full((16, 128 - tail), _NEG, jnp.float32)], axis=1)
        leaves.append((v, lane + (i * blk + j * 128)))

    def _pair(la, lb):
        (va, xa), (vb, xb) = la, lb
        c = va >= vb
        return (jnp.maximum(va, vb), jnp.where(c, xa, xb),
                jnp.minimum(va, vb), jnp.where(c, xb, xa))

    def _comb(na, nb):
        a1, ai1, a2, ai2 = na
        b1, bi1, b2, bi2 = nb
        c = a1 >= b1
        lo = jnp.minimum(a1, b1)
        loi = jnp.where(c, bi1, ai1)
        ws = jnp.where(c, a2, b2)
        wsi = jnp.where(c, ai2, bi2)
        c2 = lo >= ws
        return (jnp.maximum(a1, b1), jnp.where(c, ai1, bi1),
                jnp.maximum(lo, ws), jnp.where(c2, loi, wsi))

    nodes = [_pair(leaves[a], leaves[a + 1])
             for a in range(0, len(leaves) - 1, 2)]
    if len(leaves) % 2:
        v, x = leaves[-1]
        nodes.append((v, x, jnp.full((16, 128), _NEG, jnp.float32),
                      jnp.full((16, 128), _IMAX, jnp.int32)))
    while len(nodes) > 1:
        nxt = [_comb(nodes[a], nodes[a + 1])
               for a in range(0, len(nodes) - 1, 2)]
        if len(nodes) % 2:
            nxt.append(nodes[-1])
        nodes = nxt
    t1, t1i, t2, t2i = nodes[0]

    av = [acc_s[:, k * 128:(k + 1) * 128] for k in range(_ACC)]
    ax = [acc_i[:, k * 128:(k + 1) * 128] for k in range(_ACC)]
    b1, b1i = t1, t1i
    for k in range(_ACC):
        c = b1 > av[k]
        av[k], b1 = jnp.where(c, b1, av[k]), jnp.where(c, av[k], b1)
        ax[k], b1i = jnp.where(c, b1i, ax[k]), jnp.where(c, ax[k], b1i)
    b2, b2i = t2, t2i
    for k in range(1, _ACC):
        c = b2 > av[k]
        av[k], b2 = jnp.where(c, b2, av[k]), jnp.where(c, av[k], b2)
        ax[k], b2i = jnp.where(c, b2i, ax[k]), jnp.where(c, ax[k], b2i)
    acc_s[...] = jnp.concatenate(av, axis=1)
    acc_i[...] = jnp.concatenate(ax, axis=1)

    @pl.when(i == nblk - 1)
    def _final():
        _, mi = _extract_topk(acc_s[...], acc_i[...], _CAND)
        cand_ref[...] = jnp.concatenate(mi, axis=1)


def _rescore_kernel(idx_ref, q_ref, *refs):
    e_refs = refs[:_NCPS]
    out_i_ref, out_s_ref = refs[_NCPS], refs[_NCPS + 1]
    c = pl.program_id(0)

    @pl.when(c == 0)
    def _init():
        out_s_ref[...] = jnp.full((16, _PAD), _NEG, jnp.float32)
        out_i_ref[...] = jnp.full((16, _PAD), _IMAX, jnp.int32)

    qn = _normalize_q(q_ref[...])

    all_s, all_i = [out_s_ref[...]], [out_i_ref[...]]
    iota = jax.lax.broadcasted_iota(jnp.int32, (16, _GATH), 1)
    for j in range(_NCPS):
        e = e_refs[j][...]
        ss = jnp.sum(e * e, axis=1, keepdims=True)
        en = e * (1.0 / jnp.maximum(jnp.sqrt(ss), 1e-12))
        s = jax.lax.dot_general(
            qn, en, (((1,), (1,)), ((), ())),
            preferred_element_type=jnp.float32)
        row0 = (idx_ref[c * _NCPS + j] // _GATH) * _GATH
        all_s.append(s)
        all_i.append(row0 + iota)

    cs = jnp.concatenate(all_s, axis=1)
    ci = jnp.concatenate(all_i, axis=1)
    fs, fi = _extract_topk(cs, ci, _K)
    out_s_ref[...] = jnp.concatenate(
        fs + [jnp.full((16, _PAD - _K), _NEG, jnp.float32)], axis=1)
    out_i_ref[...] = jnp.concatenate(
        fi + [jnp.full((16, _PAD - _K), _IMAX, jnp.int32)], axis=1)


def kernel(query_embedding, evidence_embeddings, top_k):
    del top_k
    n, d = evidence_embeddings.shape
    blk = 20000 if n % 20000 == 0 else n
    nblk = n // blk

    cand = pl.pallas_call(
        functools.partial(_scan_kernel, blk=blk, nblk=nblk),
        grid=(nblk,),
        in_specs=[
            pl.BlockSpec((16, d), lambda i: (0, 0)),
            pl.BlockSpec((blk, d), lambda i: (i, 0)),
        ],
        out_specs=pl.BlockSpec((16, _CAND), lambda i: (0, 0)),
        out_shape=jax.ShapeDtypeStruct((16, _CAND), jnp.int32),
        scratch_shapes=[pltpu.VMEM((16, _ACC * 128), jnp.float32),
                        pltpu.VMEM((16, _ACC * 128), jnp.int32)],
        compiler_params=pltpu.CompilerParams(
            dimension_semantics=("arbitrary",)),
    )(query_embedding, evidence_embeddings)

    ncand = 16 * _CAND
    flat = cand.reshape(-1)

    out_i, out_s = pl.pallas_call(
        _rescore_kernel,
        grid_spec=pltpu.PrefetchScalarGridSpec(
            num_scalar_prefetch=1,
            grid=(ncand // _NCPS,),
            in_specs=[pl.BlockSpec((16, d), lambda c, s: (0, 0))] + [
                pl.BlockSpec(
                    (_GATH, d),
                    functools.partial(
                        lambda c, s, j: (s[c * _NCPS + j] // _GATH, 0), j=j))
                for j in range(_NCPS)
            ],
            out_specs=[
                pl.BlockSpec((16, _PAD), lambda c, s: (0, 0)),
                pl.BlockSpec((16, _PAD), lambda c, s: (0, 0)),
            ],
        ),
        out_shape=[
            jax.ShapeDtypeStruct((16, _PAD), jnp.int32),
            jax.ShapeDtypeStruct((16, _PAD), jnp.float32),
        ],
        compiler_params=pltpu.CompilerParams(
            dimension_semantics=("arbitrary",)),
    )(flat, query_embedding,
      *([evidence_embeddings] * _NCPS))

    return out_i[:, :_K], out_s[:, :_K]


# speedup vs baseline: 1.7285x; 1.0277x over previous
import functools

import jax
import jax.numpy as jnp
from jax.experimental import pallas as pl
from jax.experimental.pallas import tpu as pltpu

_K = 5
_CAND = 12
_HARV = 5
_PAD = 8
_NEG = float("-inf")
_IMAX = 2**30
_GATH = 40
_NCPS = 16
_ACC = 6


def _normalize_q(q):
    return q / jnp.maximum(
        jnp.sqrt(jnp.sum(q * q, axis=1, keepdims=True)), 1e-12)


def _extract_topk(cs, ci, k):
    outs_s, outs_i = [], []
    for j in range(k):
        m = jnp.max(cs, axis=1, keepdims=True)
        hit = cs == m
        idx = jnp.min(jnp.where(hit, ci, _IMAX), axis=1, keepdims=True)
        outs_s.append(m)
        outs_i.append(idx)
        if j < k - 1:
            cs = jnp.where(ci == idx, _NEG, cs)
    return outs_s, outs_i


def _scan_kernel(q_ref, e_ref, cand_ref, acc_s, acc_i, *, blk, nblk):
    i = pl.program_id(0)

    @pl.when(i == 0)
    def _init():
        acc_s[...] = jnp.full((16, _ACC * 128), _NEG, jnp.float32)
        acc_i[...] = jnp.full((16, _ACC * 128), _IMAX, jnp.int32)

    qn = _normalize_q(q_ref[...])

    e = e_ref[...]
    e_bf = e.astype(jnp.bfloat16)
    e2_bf = e_bf * e_bf
    s_raw = jax.lax.dot_general(
        qn.astype(jnp.bfloat16), e_bf, (((1,), (1,)), ((), ())),
        preferred_element_type=jnp.float32)
    ssb = jax.lax.dot_general(
        jnp.ones((16, e.shape[1]), jnp.bfloat16), e2_bf,
        (((1,), (1,)), ((), ())),
        preferred_element_type=jnp.float32)
    s_sel = s_raw * jax.lax.rsqrt(jnp.maximum(ssb, 1e-12))

    lane = jax.lax.broadcasted_iota(jnp.int32, (16, 128), 1)
    nslab = blk // 128
    tail = blk - nslab * 128
    leaves = []
    for j in range(nslab + (1 if tail else 0)):
        if j < nslab:
            v = s_sel[:, j * 128:(j + 1) * 128]
        else:
            v = jnp.concatenate(
                [s_sel[:, nslab * 128:],
                 jnp.full((16, 128 - tail), _NEG, jnp.float32)], axis=1)
        leaves.append((v, lane + (i * blk + j * 128)))

    def _pair(la, lb):
        (va, xa), (vb, xb) = la, lb
        c = va >= vb
        return (jnp.maximum(va, vb), jnp.where(c, xa, xb),
                jnp.minimum(va, vb), jnp.where(c, xb, xa))

    def _comb(na, nb):
        a1, ai1, a2, ai2 = na
        b1, bi1, b2, bi2 = nb
        c = a1 >= b1
        lo = jnp.minimum(a1, b1)
        loi = jnp.where(c, bi1, ai1)
        ws = jnp.where(c, a2, b2)
        wsi = jnp.where(c, ai2, bi2)
        c2 = lo >= ws
        return (jnp.maximum(a1, b1), jnp.where(c, ai1, bi1),
                jnp.maximum(lo, ws), jnp.where(c2, loi, wsi))

    nodes = [_pair(leaves[a], leaves[a + 1])
             for a in range(0, len(leaves) - 1, 2)]
    if len(leaves) % 2:
        v, x = leaves[-1]
        nodes.append((v, x, jnp.full((16, 128), _NEG, jnp.float32),
                      jnp.full((16, 128), _IMAX, jnp.int32)))
    while len(nodes) > 1:
        nxt = [_comb(nodes[a], nodes[a + 1])
               for a in range(0, len(nodes) - 1, 2)]
        if len(nodes) % 2:
            nxt.append(nodes[-1])
        nodes = nxt
    t1, t1i, t2, t2i = nodes[0]

    av = [acc_s[:, k * 128:(k + 1) * 128] for k in range(_ACC)]
    ax = [acc_i[:, k * 128:(k + 1) * 128] for k in range(_ACC)]
    b1, b1i = t1, t1i
    for k in range(_ACC):
        c = b1 > av[k]
        av[k], b1 = jnp.where(c, b1, av[k]), jnp.where(c, av[k], b1)
        ax[k], b1i = jnp.where(c, b1i, ax[k]), jnp.where(c, ax[k], b1i)
    b2, b2i = t2, t2i
    for k in range(1, _ACC):
        c = b2 > av[k]
        av[k], b2 = jnp.where(c, b2, av[k]), jnp.where(c, av[k], b2)
        ax[k], b2i = jnp.where(c, b2i, ax[k]), jnp.where(c, ax[k], b2i)
    acc_s[...] = jnp.concatenate(av, axis=1)
    acc_i[...] = jnp.concatenate(ax, axis=1)

    @pl.when(i == nblk - 1)
    def _final():
        _, mi = _extract_topk(acc_s[...], acc_i[...], _CAND)
        cand_ref[...] = jnp.concatenate(mi, axis=1)


def _rescore_kernel(idx_ref, q_ref, *refs):
    e_refs = refs[:_NCPS]
    out_i_ref, out_s_ref = refs[_NCPS], refs[_NCPS + 1]
    c = pl.program_id(0)

    @pl.when(c == 0)
    def _init():
        out_s_ref[...] = jnp.full((16, _PAD), _NEG, jnp.float32)
        out_i_ref[...] = jnp.full((16, _PAD), _IMAX, jnp.int32)

    qn = _normalize_q(q_ref[...])

    all_s, all_i = [out_s_ref[...]], [out_i_ref[...]]
    iota = jax.lax.broadcasted_iota(jnp.int32, (16, _GATH), 1)
    for j in range(_NCPS):
        e = e_refs[j][...]
        ss = jnp.sum(e * e, axis=1, keepdims=True)
        en = e * (1.0 / jnp.maximum(jnp.sqrt(ss), 1e-12))
        s = jax.lax.dot_general(
            qn, en, (((1,), (1,)), ((), ())),
            preferred_element_type=jnp.float32)
        row0 = (idx_ref[c * _NCPS + j] // _GATH) * _GATH
        all_s.append(s)
        all_i.append(row0 + iota)

    cs = jnp.concatenate(all_s, axis=1)
    ci = jnp.concatenate(all_i, axis=1)
    fs, fi = _extract_topk(cs, ci, _K)
    out_s_ref[...] = jnp.concatenate(
        fs + [jnp.full((16, _PAD - _K), _NEG, jnp.float32)], axis=1)
    out_i_ref[...] = jnp.concatenate(
        fi + [jnp.full((16, _PAD - _K), _IMAX, jnp.int32)], axis=1)


def kernel(query_embedding, evidence_embeddings, top_k):
    del top_k
    n, d = evidence_embeddings.shape
    blk = 25000 if n % 25000 == 0 else n
    nblk = n // blk

    cand = pl.pallas_call(
        functools.partial(_scan_kernel, blk=blk, nblk=nblk),
        grid=(nblk,),
        in_specs=[
            pl.BlockSpec((16, d), lambda i: (0, 0)),
            pl.BlockSpec((blk, d), lambda i: (i, 0)),
        ],
        out_specs=pl.BlockSpec((16, _CAND), lambda i: (0, 0)),
        out_shape=jax.ShapeDtypeStruct((16, _CAND), jnp.int32),
        scratch_shapes=[pltpu.VMEM((16, _ACC * 128), jnp.float32),
                        pltpu.VMEM((16, _ACC * 128), jnp.int32)],
        compiler_params=pltpu.CompilerParams(
            dimension_semantics=("arbitrary",)),
    )(query_embedding, evidence_embeddings)

    ncand = 16 * _CAND
    flat = cand.reshape(-1)

    out_i, out_s = pl.pallas_call(
        _rescore_kernel,
        grid_spec=pltpu.PrefetchScalarGridSpec(
            num_scalar_prefetch=1,
            grid=(ncand // _NCPS,),
            in_specs=[pl.BlockSpec((16, d), lambda c, s: (0, 0))] + [
                pl.BlockSpec(
                    (_GATH, d),
                    functools.partial(
                        lambda c, s, j: (s[c * _NCPS + j] // _GATH, 0), j=j))
                for j in range(_NCPS)
            ],
            out_specs=[
                pl.BlockSpec((16, _PAD), lambda c, s: (0, 0)),
                pl.BlockSpec((16, _PAD), lambda c, s: (0, 0)),
            ],
        ),
        out_shape=[
            jax.ShapeDtypeStruct((16, _PAD), jnp.int32),
            jax.ShapeDtypeStruct((16, _PAD), jnp.float32),
        ],
        compiler_params=pltpu.CompilerParams(
            dimension_semantics=("arbitrary",)),
    )(flat, query_embedding,
      *([evidence_embeddings] * _NCPS))

    return out_i[:, :_K], out_s[:, :_K]


# speedup vs baseline: 1.7963x; 1.0392x over previous
import functools

import jax
import jax.numpy as jnp
from jax.experimental import pallas as pl
from jax.experimental.pallas import tpu as pltpu

_K = 5
_CAND = 12
_HARV = 5
_PAD = 8
_NEG = float("-inf")
_IMAX = 2**30
_GATH = 40
_NCPS = 16
_ACC = 6


def _normalize_q(q):
    return q / jnp.maximum(
        jnp.sqrt(jnp.sum(q * q, axis=1, keepdims=True)), 1e-12)


def _extract_topk(cs, ci, k):
    outs_s, outs_i = [], []
    for j in range(k):
        m = jnp.max(cs, axis=1, keepdims=True)
        hit = cs == m
        idx = jnp.min(jnp.where(hit, ci, _IMAX), axis=1, keepdims=True)
        outs_s.append(m)
        outs_i.append(idx)
        if j < k - 1:
            cs = jnp.where(ci == idx, _NEG, cs)
    return outs_s, outs_i


def _scan_kernel(q_ref, e_ref, cand_ref, acc_s, acc_i, *, blk, nblk):
    i = pl.program_id(0)

    @pl.when(i == 0)
    def _init():
        acc_s[...] = jnp.full((16, _ACC * 128), _NEG, jnp.float32)
        acc_i[...] = jnp.full((16, _ACC * 128), _IMAX, jnp.int32)

    qn = _normalize_q(q_ref[...])

    e = e_ref[...]
    e_bf = e.astype(jnp.bfloat16)
    e2_bf = e_bf * e_bf
    s_raw = jax.lax.dot_general(
        qn.astype(jnp.bfloat16), e_bf, (((1,), (1,)), ((), ())),
        preferred_element_type=jnp.float32)
    ssb = jax.lax.dot_general(
        jnp.ones((16, e.shape[1]), jnp.bfloat16), e2_bf,
        (((1,), (1,)), ((), ())),
        preferred_element_type=jnp.float32)
    s_sel = s_raw * jax.lax.rsqrt(jnp.maximum(ssb, 1e-12))

    lane = jax.lax.broadcasted_iota(jnp.int32, (16, 128), 1)
    nslab = blk // 128
    tail = blk - nslab * 128
    leaves = []
    for j in range(nslab + (1 if tail else 0)):
        if j < nslab:
            v = s_sel[:, j * 128:(j + 1) * 128]
        else:
            v = jnp.concatenate(
                [s_sel[:, nslab * 128:],
                 jnp.full((16, 128 - tail), _NEG, jnp.float32)], axis=1)
        leaves.append((v, lane + (i * blk + j * 128)))

    def _pair(la, lb):
        (va, xa), (vb, xb) = la, lb
        c = va >= vb
        return (jnp.maximum(va, vb), jnp.where(c, xa, xb),
                jnp.minimum(va, vb), jnp.where(c, xb, xa))

    def _comb(na, nb):
        a1, ai1, a2, ai2 = na
        b1, bi1, b2, bi2 = nb
        c = a1 >= b1
        lo = jnp.minimum(a1, b1)
        loi = jnp.where(c, bi1, ai1)
        ws = jnp.where(c, a2, b2)
        wsi = jnp.where(c, ai2, bi2)
        c2 = lo >= ws
        return (jnp.maximum(a1, b1), jnp.where(c, ai1, bi1),
                jnp.maximum(lo, ws), jnp.where(c2, loi, wsi))

    nodes = [_pair(leaves[a], leaves[a + 1])
             for a in range(0, len(leaves) - 1, 2)]
    if len(leaves) % 2:
        v, x = leaves[-1]
        nodes.append((v, x, jnp.full((16, 128), _NEG, jnp.float32),
                      jnp.full((16, 128), _IMAX, jnp.int32)))
    while len(nodes) > 1:
        nxt = [_comb(nodes[a], nodes[a + 1])
               for a in range(0, len(nodes) - 1, 2)]
        if len(nodes) % 2:
            nxt.append(nodes[-1])
        nodes = nxt
    t1, t1i, t2, t2i = nodes[0]

    av = [acc_s[:, k * 128:(k + 1) * 128] for k in range(_ACC)]
    ax = [acc_i[:, k * 128:(k + 1) * 128] for k in range(_ACC)]
    b1, b1i = t1, t1i
    for k in range(_ACC):
        c = b1 > av[k]
        av[k], b1 = jnp.where(c, b1, av[k]), jnp.where(c, av[k], b1)
        ax[k], b1i = jnp.where(c, b1i, ax[k]), jnp.where(c, ax[k], b1i)
    b2, b2i = t2, t2i
    for k in range(1, _ACC):
        c = b2 > av[k]
        av[k], b2 = jnp.where(c, b2, av[k]), jnp.where(c, av[k], b2)
        ax[k], b2i = jnp.where(c, b2i, ax[k]), jnp.where(c, ax[k], b2i)
    acc_s[...] = jnp.concatenate(av, axis=1)
    acc_i[...] = jnp.concatenate(ax, axis=1)

    @pl.when(i == nblk - 1)
    def _final():
        _, mi = _extract_topk(acc_s[...], acc_i[...], _CAND)
        cand_ref[...] = jnp.concatenate(mi, axis=1)


def _rescore_kernel(idx_ref, q_ref, *refs):
    e_refs = refs[:_NCPS]
    out_i_ref, out_s_ref = refs[_NCPS], refs[_NCPS + 1]
    c = pl.program_id(0)

    @pl.when(c == 0)
    def _init():
        out_s_ref[...] = jnp.full((16, _PAD), _NEG, jnp.float32)
        out_i_ref[...] = jnp.full((16, _PAD), _IMAX, jnp.int32)

    qn = _normalize_q(q_ref[...])

    all_s, all_i = [out_s_ref[...]], [out_i_ref[...]]
    iota = jax.lax.broadcasted_iota(jnp.int32, (16, _GATH), 1)
    for j in range(_NCPS):
        e = e_refs[j][...]
        ss = jnp.sum(e * e, axis=1, keepdims=True)
        en = e * (1.0 / jnp.maximum(jnp.sqrt(ss), 1e-12))
        s = jax.lax.dot_general(
            qn, en, (((1,), (1,)), ((), ())),
            preferred_element_type=jnp.float32)
        row0 = (idx_ref[c * _NCPS + j] // _GATH) * _GATH
        all_s.append(s)
        all_i.append(row0 + iota)

    cs = jnp.concatenate(all_s, axis=1)
    ci = jnp.concatenate(all_i, axis=1)
    fs, fi = _extract_topk(cs, ci, _K)
    out_s_ref[...] = jnp.concatenate(
        fs + [jnp.full((16, _PAD - _K), _NEG, jnp.float32)], axis=1)
    out_i_ref[...] = jnp.concatenate(
        fi + [jnp.full((16, _PAD - _K), _IMAX, jnp.int32)], axis=1)


def kernel(query_embedding, evidence_embeddings, top_k):
    del top_k
    n, d = evidence_embeddings.shape
    blk = 40000 if n % 40000 == 0 else n
    nblk = n // blk

    cand = pl.pallas_call(
        functools.partial(_scan_kernel, blk=blk, nblk=nblk),
        grid=(nblk,),
        in_specs=[
            pl.BlockSpec((16, d), lambda i: (0, 0)),
            pl.BlockSpec((blk, d), lambda i: (i, 0)),
        ],
        out_specs=pl.BlockSpec((16, _CAND), lambda i: (0, 0)),
        out_shape=jax.ShapeDtypeStruct((16, _CAND), jnp.int32),
        scratch_shapes=[pltpu.VMEM((16, _ACC * 128), jnp.float32),
                        pltpu.VMEM((16, _ACC * 128), jnp.int32)],
        compiler_params=pltpu.CompilerParams(
            dimension_semantics=("arbitrary",)),
    )(query_embedding, evidence_embeddings)

    ncand = 16 * _CAND
    flat = cand.reshape(-1)

    out_i, out_s = pl.pallas_call(
        _rescore_kernel,
        grid_spec=pltpu.PrefetchScalarGridSpec(
            num_scalar_prefetch=1,
            grid=(ncand // _NCPS,),
            in_specs=[pl.BlockSpec((16, d), lambda c, s: (0, 0))] + [
                pl.BlockSpec(
                    (_GATH, d),
                    functools.partial(
                        lambda c, s, j: (s[c * _NCPS + j] // _GATH, 0), j=j))
                for j in range(_NCPS)
            ],
            out_specs=[
                pl.BlockSpec((16, _PAD), lambda c, s: (0, 0)),
                pl.BlockSpec((16, _PAD), lambda c, s: (0, 0)),
            ],
        ),
        out_shape=[
            jax.ShapeDtypeStruct((16, _PAD), jnp.int32),
            jax.ShapeDtypeStruct((16, _PAD), jnp.float32),
        ],
        compiler_params=pltpu.CompilerParams(
            dimension_semantics=("arbitrary",)),
    )(flat, query_embedding,
      *([evidence_embeddings] * _NCPS))

    return out_i[:, :_K], out_s[:, :_K]
